# Initial kernel scaffold; baseline (speedup 1.0000x reference)
import jax
import jax.numpy as jnp
from jax.experimental import pallas as pl


def kernel(x, edge_index, target_key, params):
    raise NotImplementedError("write your pallas kernel here")


# speedup vs baseline: 2.2243x; 2.2243x over previous
import functools

import jax
import jax.numpy as jnp
from jax import lax
from jax.experimental import pallas as pl
from jax.experimental.pallas import tpu as pltpu
from jax.experimental.pallas import tpu_sc as plsc

N = 10000
NP = 10112
DUMMY = 10000
E = 160000
UNITS = 32
EG = 40
EC = 128
EPAD = UNITS * EG * EC - E
ROWS_PT = NP // 16
MB = 2528

_GROUPS = [[0, 1, 2, 3], [4, 5], [6, 7], [8, 9], [10, 11], [12, 13]]


def _cfg(cout):
    if cout >= 256:
        return cout // 128, 128, False
    return 1, cout, True


def _matmul(h, w, nchunks, cc):
    cin = h.shape[1]

    def body(h_ref, w_ref, o_ref):
        o_ref[0] = jnp.dot(h_ref[...], w_ref[...],
                           preferred_element_type=jnp.float32)

    return pl.pallas_call(
        body,
        grid=(nchunks, NP // MB),
        in_specs=[
            pl.BlockSpec((MB, cin), lambda j, i: (i, 0)),
            pl.BlockSpec((cin, cc), lambda j, i: (0, j)),
        ],
        out_specs=pl.BlockSpec((1, MB, cc), lambda j, i: (j, i, 0)),
        out_shape=jax.ShapeDtypeStruct((nchunks, NP, cc), jnp.float32),
    )(h, w)


def _sc_segment_sum(hw, srcp, dstp, zrows, nchunks, cc, edge_split):
    cps = 1 if edge_split else nchunks // 2
    n_out = 2 if edge_split else nchunks
    mesh = plsc.VectorSubcoreMesh(core_axis_name="c", subcore_axis_name="s",
                                  num_cores=2, num_subcores=16)

    @functools.partial(
        pl.kernel,
        out_type=jax.ShapeDtypeStruct((n_out, NP, cc), jnp.float32),
        mesh=mesh,
        compiler_params=pltpu.CompilerParams(use_tc_tiling_on_sc=False),
        scratch_types=[
            pltpu.VMEM((EG, EC), jnp.int32),
            pltpu.VMEM((EG, EC), jnp.int32),
            pltpu.VMEM((EC, cc), jnp.float32),
            pltpu.VMEM_SHARED((NP, cc), jnp.float32),
            pltpu.SemaphoreType.DMA,
        ],
    )
    def k(hw_hbm, srcp_hbm, dstp_hbm, zr_hbm, out_hbm,
          src_v, dst_v, buf_v, agg_sh, sem):
        c = lax.axis_index("c")
        s = lax.axis_index("s")
        for ci in range(cps):
            chunk = c if edge_split else c * cps + ci
            pltpu.sync_copy(zr_hbm, agg_sh.at[pl.ds(s * ROWS_PT, ROWS_PT)])
            plsc.subcore_barrier()
            for ui in range(1 if edge_split else 2):
                u = c * 16 + s if edge_split else s + 16 * ui
                pltpu.sync_copy(srcp_hbm.at[u], src_v)
                pltpu.sync_copy(dstp_hbm.at[u], dst_v)
                hw_c = hw_hbm.at[0 if edge_split else chunk]

                def body(g, carry):
                    pltpu.async_copy(hw_c.at[src_v.at[g]], buf_v, sem).wait()
                    pltpu.sync_copy(buf_v, agg_sh.at[dst_v.at[g]], add=True)
                    return carry

                lax.fori_loop(0, EG, body, 0)
            plsc.subcore_barrier()
            pltpu.sync_copy(
                agg_sh.at[pl.ds(s * ROWS_PT, ROWS_PT)],
                out_hbm.at[chunk].at[pl.ds(s * ROWS_PT, ROWS_PT)])

    return k(hw, srcp, dstp, zrows)


def _bn_elu(agg, gamma, beta, edge_split):
    cout = gamma.shape[0]

    def math(a, g, b):
        mu = jnp.mean(a, axis=0, keepdims=True)
        xc = a - mu
        var = jnp.mean(xc * xc, axis=0, keepdims=True)
        hn = g * (xc * lax.rsqrt(var + 1e-5)) + b
        return jnp.where(hn > 0, hn, jnp.exp(hn) - 1.0)

    if edge_split:
        def body(a_ref, g_ref, b_ref, o_ref):
            o_ref[...] = math(a_ref[0] + a_ref[1], g_ref[0], b_ref[0])

        return pl.pallas_call(
            body,
            grid=(1,),
            in_specs=[
                pl.BlockSpec((2, N, cout), lambda j: (0, 0, 0)),
                pl.BlockSpec((1, 1, cout), lambda j: (0, 0, 0)),
                pl.BlockSpec((1, 1, cout), lambda j: (0, 0, 0)),
            ],
            out_specs=pl.BlockSpec((N, cout), lambda j: (0, 0)),
            out_shape=jax.ShapeDtypeStruct((NP, cout), jnp.float32),
        )(agg, gamma.reshape(1, 1, cout), beta.reshape(1, 1, cout))

    nchunks, cc = agg.shape[0], agg.shape[2]

    def body(a_ref, g_ref, b_ref, o_ref):
        o_ref[...] = math(a_ref[0], g_ref[0], b_ref[0])

    return pl.pallas_call(
        body,
        grid=(nchunks,),
        in_specs=[
            pl.BlockSpec((1, N, cc), lambda j: (j, 0, 0)),
            pl.BlockSpec((1, 1, cc), lambda j: (j, 0, 0)),
            pl.BlockSpec((1, 1, cc), lambda j: (j, 0, 0)),
        ],
        out_specs=pl.BlockSpec((N, cc), lambda j: (0, j)),
        out_shape=jax.ShapeDtypeStruct((NP, cout), jnp.float32),
    )(agg, gamma.reshape(nchunks, 1, cc), beta.reshape(nchunks, 1, cc))


def _cls_prune(h, cw, cb, last):
    C = h.shape[1]
    cw2 = cw.reshape(1, C)
    cb2 = cb.reshape(1, 1)

    if not last:
        NB = 2000

        def body(h_ref, w_ref, b_ref, lg_ref, o_ref):
            hh = h_ref[...]
            lg = jnp.sum(hh * w_ref[...], axis=1, keepdims=True) + b_ref[...]
            lg_ref[...] = lg
            o_ref[...] = hh * (lg > 0).astype(jnp.float32)

        return pl.pallas_call(
            body,
            grid=(N // NB,),
            in_specs=[
                pl.BlockSpec((NB, C), lambda i: (i, 0)),
                pl.BlockSpec((1, C), lambda i: (0, 0)),
                pl.BlockSpec((1, 1), lambda i: (0, 0)),
            ],
            out_specs=[
                pl.BlockSpec((NB, 1), lambda i: (i, 0)),
                pl.BlockSpec((NB, C), lambda i: (i, 0)),
            ],
            out_shape=[
                jax.ShapeDtypeStruct((N, 1), jnp.float32),
                jax.ShapeDtypeStruct((NP, C), jnp.float32),
            ],
        )(h, cw2, cb2)

    def body(h_ref, w_ref, b_ref, lg_ref, o_ref):
        hh = h_ref[...]
        lg = jnp.sum(hh * w_ref[...], axis=1, keepdims=True) + b_ref[...]
        keep = (lg > 0).astype(jnp.float32)
        keep = jnp.where(jnp.sum(keep) > 0, keep, jnp.ones_like(keep))
        lg_ref[...] = lg
        o_ref[...] = hh * keep

    return pl.pallas_call(
        body,
        grid=(1,),
        in_specs=[
            pl.BlockSpec((N, C), lambda i: (0, 0)),
            pl.BlockSpec((1, C), lambda i: (0, 0)),
            pl.BlockSpec((1, 1), lambda i: (0, 0)),
        ],
        out_specs=[
            pl.BlockSpec((N, 1), lambda i: (0, 0)),
            pl.BlockSpec((N, C), lambda i: (0, 0)),
        ],
        out_shape=[
            jax.ShapeDtypeStruct((N, 1), jnp.float32),
            jax.ShapeDtypeStruct((NP, C), jnp.float32),
        ],
    )(h, cw2, cb2)


def _target_mask(tkp, zr, ones):
    mesh = plsc.VectorSubcoreMesh(core_axis_name="c", subcore_axis_name="s",
                                  num_cores=2, num_subcores=16)
    ngroups = tkp.shape[0]

    @functools.partial(
        pl.kernel,
        out_type=jax.ShapeDtypeStruct((NP,), jnp.int32),
        mesh=mesh,
        scratch_types=[
            pltpu.VMEM((ngroups, EC), jnp.int32),
            pltpu.VMEM((EC,), jnp.int32),
            pltpu.VMEM((2528,), jnp.int32),
        ],
    )
    def k(tk_hbm, zr_hbm, ones_hbm, out_hbm, tk_v, ones_v, zr_v):
        c = lax.axis_index("c")
        s = lax.axis_index("s")

        @pl.when(jnp.logical_and(c == 0, s == 0))
        def _():
            pltpu.sync_copy(zr_hbm, zr_v)
            for j in range(NP // 2528):
                pltpu.sync_copy(zr_v, out_hbm.at[pl.ds(j * 2528, 2528)])
            pltpu.sync_copy(tk_hbm, tk_v)
            pltpu.sync_copy(ones_hbm, ones_v)

            def body(g, carry):
                pltpu.sync_copy(ones_v, out_hbm.at[tk_v.at[g]])
                return carry

            lax.fori_loop(0, ngroups, body, 0)

    return k(tkp, zr, ones)


def kernel(x, edge_index, target_key, params):
    src = edge_index[0].astype(jnp.int32)
    dst = edge_index[1].astype(jnp.int32)
    srcp = jnp.concatenate(
        [src, jnp.zeros((EPAD,), jnp.int32)]).reshape(UNITS, EG, EC)
    dstp = jnp.concatenate(
        [dst, jnp.full((EPAD,), DUMMY, jnp.int32)]).reshape(UNITS, EG, EC)

    nk = target_key.shape[0]
    tk_pad = (-nk) % EC
    tkp = jnp.concatenate(
        [target_key.astype(jnp.int32),
         jnp.full((tk_pad,), DUMMY, jnp.int32)]).reshape(-1, EC)
    tgt = _target_mask(tkp, jnp.zeros((2528,), jnp.int32),
                       jnp.ones((EC,), jnp.int32))
    target = tgt[:N].astype(bool)

    h = jnp.pad(x, ((0, NP - N), (0, 0)))
    out_cls = []
    for bi, group in enumerate(_GROUPS):
        for li in group:
            w, _b, gamma, beta = params["layers"][li]
            cout = w.shape[1]
            nchunks, cc, edge_split = _cfg(cout)
            hw = _matmul(h, w, nchunks, cc)
            agg = _sc_segment_sum(hw, srcp, dstp,
                                  jnp.zeros((ROWS_PT, cc), jnp.float32),
                                  nchunks, cc, edge_split)
            h = _bn_elu(agg, gamma, beta, edge_split)
        cw, cb = params["cls"][bi]
        logits, h = _cls_prune(h, cw, cb, last=(bi == len(_GROUPS) - 1))
        out_cls.append(logits)

    return (jnp.stack(out_cls),
            jnp.stack([target] * len(_GROUPS)),
            h[:N])
